# initial kernel scaffold (unmeasured)
import jax
import jax.numpy as jnp
from jax import lax
from jax.experimental import pallas as pl
from jax.experimental.pallas import tpu as pltpu


def kernel(
    x,
):
    def body(*refs):
        pass

    out_shape = jax.ShapeDtypeStruct(..., jnp.float32)
    return pl.pallas_call(body, out_shape=out_shape)(...)


# baseline (device time: 109657 ns/iter reference)
import jax
import jax.numpy as jnp
from jax import lax
from jax.experimental import pallas as pl
from jax.experimental.pallas import tpu as pltpu


def kernel(x):
    m, n = x.shape
    half = m // 2

    def body(x_ref, out_ref, s1_send, s1_recv, red, s2_recv,
             sem1_s, sem1_r, sem2_s, sem2_r):
        my_x = lax.axis_index("x")
        my_y = lax.axis_index("y")
        h = my_x ^ my_y
        oh = 1 - h

        barrier_sem = pltpu.get_barrier_semaphore()
        for nbr in ((1 - my_x, my_y), (my_x, 1 - my_y)):
            pl.semaphore_signal(
                barrier_sem, inc=1,
                device_id=nbr, device_id_type=pl.DeviceIdType.MESH,
            )
        pl.semaphore_wait(barrier_sem, 2)

        s1_send[...] = x_ref[pl.ds(oh * half, half), :].astype(jnp.bfloat16)
        rdma1 = pltpu.make_async_remote_copy(
            src_ref=s1_send,
            dst_ref=s1_recv,
            send_sem=sem1_s,
            recv_sem=sem1_r,
            device_id=(1 - my_x, my_y),
            device_id_type=pl.DeviceIdType.MESH,
        )
        rdma1.start()
        rdma1.wait()

        mine = x_ref[pl.ds(h * half, half), :]
        red[...] = (mine + s1_recv[...].astype(jnp.float32)).astype(jnp.bfloat16)

        rdma2 = pltpu.make_async_remote_copy(
            src_ref=red,
            dst_ref=s2_recv,
            send_sem=sem2_s,
            recv_sem=sem2_r,
            device_id=(my_x, 1 - my_y),
            device_id_type=pl.DeviceIdType.MESH,
        )
        rdma2.start()
        rdma2.wait()

        out_ref[pl.ds(h * half, half), :] = red[...].astype(jnp.float32)
        out_ref[pl.ds(oh * half, half), :] = s2_recv[...].astype(jnp.float32)

    return pl.pallas_call(
        body,
        out_shape=jax.ShapeDtypeStruct((m, n), jnp.float32),
        in_specs=[pl.BlockSpec(memory_space=pltpu.VMEM)],
        out_specs=pl.BlockSpec(memory_space=pltpu.VMEM),
        scratch_shapes=[
            pltpu.VMEM((half, n), jnp.bfloat16),
            pltpu.VMEM((half, n), jnp.bfloat16),
            pltpu.VMEM((half, n), jnp.bfloat16),
            pltpu.VMEM((half, n), jnp.bfloat16),
            pltpu.SemaphoreType.DMA,
            pltpu.SemaphoreType.DMA,
            pltpu.SemaphoreType.DMA,
            pltpu.SemaphoreType.DMA,
        ],
        compiler_params=pltpu.CompilerParams(collective_id=0),
    )(x)


# device time: 68928 ns/iter; 1.5909x vs baseline; 1.5909x over previous
import jax
import jax.numpy as jnp
from jax import lax
from jax.experimental import pallas as pl
from jax.experimental.pallas import tpu as pltpu

NCHUNK = 8


def kernel(x):
    m, n = x.shape
    half = m // 2
    rows = half // NCHUNK

    def body(x_ref, out_ref, s1_send, s1_recv, red, s2_recv,
             sems1_s, sems1_r, sems2_s, sems2_r):
        my_x = lax.axis_index("x")
        my_y = lax.axis_index("y")
        h = my_x ^ my_y
        oh = 1 - h

        barrier_sem = pltpu.get_barrier_semaphore()
        for nbr in ((1 - my_x, my_y), (my_x, 1 - my_y)):
            pl.semaphore_signal(
                barrier_sem, inc=1,
                device_id=nbr, device_id_type=pl.DeviceIdType.MESH,
            )
        pl.semaphore_wait(barrier_sem, 2)

        s1_send[...] = x_ref[pl.ds(oh * half, half), :].astype(jnp.bfloat16)
        rdma1 = []
        for c in range(NCHUNK):
            r = pltpu.make_async_remote_copy(
                src_ref=s1_send.at[pl.ds(c * rows, rows), :],
                dst_ref=s1_recv.at[pl.ds(c * rows, rows), :],
                send_sem=sems1_s.at[c],
                recv_sem=sems1_r.at[c],
                device_id=(1 - my_x, my_y),
                device_id_type=pl.DeviceIdType.MESH,
            )
            r.start()
            rdma1.append(r)

        rdma2 = []
        for c in range(NCHUNK):
            rdma1[c].wait_recv()
            mine = x_ref[pl.ds(h * half + c * rows, rows), :]
            red[pl.ds(c * rows, rows), :] = (
                mine + s1_recv[pl.ds(c * rows, rows), :].astype(jnp.float32)
            ).astype(jnp.bfloat16)
            r = pltpu.make_async_remote_copy(
                src_ref=red.at[pl.ds(c * rows, rows), :],
                dst_ref=s2_recv.at[pl.ds(c * rows, rows), :],
                send_sem=sems2_s.at[c],
                recv_sem=sems2_r.at[c],
                device_id=(my_x, 1 - my_y),
                device_id_type=pl.DeviceIdType.MESH,
            )
            r.start()
            rdma2.append(r)
            out_ref[pl.ds(h * half + c * rows, rows), :] = (
                red[pl.ds(c * rows, rows), :].astype(jnp.float32)
            )

        for c in range(NCHUNK):
            rdma2[c].wait_recv()
            out_ref[pl.ds(oh * half + c * rows, rows), :] = (
                s2_recv[pl.ds(c * rows, rows), :].astype(jnp.float32)
            )

        for c in range(NCHUNK):
            rdma1[c].wait_send()
            rdma2[c].wait_send()

    return pl.pallas_call(
        body,
        out_shape=jax.ShapeDtypeStruct((m, n), jnp.float32),
        in_specs=[pl.BlockSpec(memory_space=pltpu.VMEM)],
        out_specs=pl.BlockSpec(memory_space=pltpu.VMEM),
        scratch_shapes=[
            pltpu.VMEM((half, n), jnp.bfloat16),
            pltpu.VMEM((half, n), jnp.bfloat16),
            pltpu.VMEM((half, n), jnp.bfloat16),
            pltpu.VMEM((half, n), jnp.bfloat16),
            pltpu.SemaphoreType.DMA((NCHUNK,)),
            pltpu.SemaphoreType.DMA((NCHUNK,)),
            pltpu.SemaphoreType.DMA((NCHUNK,)),
            pltpu.SemaphoreType.DMA((NCHUNK,)),
        ],
        compiler_params=pltpu.CompilerParams(collective_id=0),
    )(x)


# device time: 66007 ns/iter; 1.6613x vs baseline; 1.0443x over previous
import jax
import jax.numpy as jnp
from jax import lax
from jax.experimental import pallas as pl
from jax.experimental.pallas import tpu as pltpu

NCHUNK = 8


def kernel(x):
    m, n = x.shape
    half = m // 2
    rows = half // NCHUNK

    def body(x_ref, out_ref, s1_send, s1_recv,
             sems1_s, sems1_r, sems2_s, sems2_r):
        my_x = lax.axis_index("x")
        my_y = lax.axis_index("y")
        h = my_x ^ my_y
        oh = 1 - h

        barrier_sem = pltpu.get_barrier_semaphore()
        for nbr in ((1 - my_x, my_y), (my_x, 1 - my_y)):
            pl.semaphore_signal(
                barrier_sem, inc=1,
                device_id=nbr, device_id_type=pl.DeviceIdType.MESH,
            )
        pl.semaphore_wait(barrier_sem, 2)

        rdma1 = []
        for c in range(NCHUNK):
            s1_send[pl.ds(c * rows, rows), :] = (
                x_ref[pl.ds(oh * half + c * rows, rows), :].astype(jnp.bfloat16)
            )
            r = pltpu.make_async_remote_copy(
                src_ref=s1_send.at[pl.ds(c * rows, rows), :],
                dst_ref=s1_recv.at[pl.ds(c * rows, rows), :],
                send_sem=sems1_s.at[c],
                recv_sem=sems1_r.at[c],
                device_id=(1 - my_x, my_y),
                device_id_type=pl.DeviceIdType.MESH,
            )
            r.start()
            rdma1.append(r)

        rdma2 = []
        for c in range(NCHUNK):
            rdma1[c].wait_recv()
            off = h * half + c * rows
            mine = x_ref[pl.ds(off, rows), :]
            out_ref[pl.ds(off, rows), :] = (
                mine + s1_recv[pl.ds(c * rows, rows), :].astype(jnp.float32)
            ).astype(jnp.bfloat16)
            r = pltpu.make_async_remote_copy(
                src_ref=out_ref.at[pl.ds(off, rows), :],
                dst_ref=out_ref.at[pl.ds(off, rows), :],
                send_sem=sems2_s.at[c],
                recv_sem=sems2_r.at[c],
                device_id=(my_x, 1 - my_y),
                device_id_type=pl.DeviceIdType.MESH,
            )
            r.start()
            rdma2.append(r)

        for c in range(NCHUNK):
            rdma2[c].wait_recv()
        for c in range(NCHUNK):
            rdma1[c].wait_send()
            rdma2[c].wait_send()

    return pl.pallas_call(
        body,
        out_shape=jax.ShapeDtypeStruct((m, n), jnp.bfloat16),
        in_specs=[pl.BlockSpec(memory_space=pltpu.VMEM)],
        out_specs=pl.BlockSpec(memory_space=pltpu.VMEM),
        scratch_shapes=[
            pltpu.VMEM((half, n), jnp.bfloat16),
            pltpu.VMEM((half, n), jnp.bfloat16),
            pltpu.SemaphoreType.DMA((NCHUNK,)),
            pltpu.SemaphoreType.DMA((NCHUNK,)),
            pltpu.SemaphoreType.DMA((NCHUNK,)),
            pltpu.SemaphoreType.DMA((NCHUNK,)),
        ],
        compiler_params=pltpu.CompilerParams(collective_id=0),
    )(x)


# device time: 61544 ns/iter; 1.7818x vs baseline; 1.0725x over previous
import jax
import jax.numpy as jnp
from jax import lax
from jax.experimental import pallas as pl
from jax.experimental.pallas import tpu as pltpu

NCHUNK = 8


def kernel(x):
    m, n = x.shape
    half = m // 2
    rows = half // NCHUNK

    def body(x_ref, out_ref, xa_oh, xa_h, s1_send, s1_recv, red,
             sems_in_oh, sems_in_h, sems_out,
             sems1_s, sems1_r, sems2_s, sems2_r):
        my_x = lax.axis_index("x")
        my_y = lax.axis_index("y")
        h = my_x ^ my_y
        oh = 1 - h

        in_oh, in_h = [], []
        for c in range(NCHUNK):
            cp = pltpu.make_async_copy(
                x_ref.at[pl.ds(oh * half + c * rows, rows), :],
                xa_oh.at[pl.ds(c * rows, rows), :],
                sems_in_oh.at[c],
            )
            cp.start()
            in_oh.append(cp)
            cp = pltpu.make_async_copy(
                x_ref.at[pl.ds(h * half + c * rows, rows), :],
                xa_h.at[pl.ds(c * rows, rows), :],
                sems_in_h.at[c],
            )
            cp.start()
            in_h.append(cp)

        barrier_sem = pltpu.get_barrier_semaphore()
        for nbr in ((1 - my_x, my_y), (my_x, 1 - my_y)):
            pl.semaphore_signal(
                barrier_sem, inc=1,
                device_id=nbr, device_id_type=pl.DeviceIdType.MESH,
            )
        pl.semaphore_wait(barrier_sem, 2)

        rdma1 = []
        for c in range(NCHUNK):
            in_oh[c].wait()
            s1_send[pl.ds(c * rows, rows), :] = (
                xa_oh[pl.ds(c * rows, rows), :].astype(jnp.bfloat16)
            )
            r = pltpu.make_async_remote_copy(
                src_ref=s1_send.at[pl.ds(c * rows, rows), :],
                dst_ref=s1_recv.at[pl.ds(c * rows, rows), :],
                send_sem=sems1_s.at[c],
                recv_sem=sems1_r.at[c],
                device_id=(1 - my_x, my_y),
                device_id_type=pl.DeviceIdType.MESH,
            )
            r.start()
            rdma1.append(r)

        rdma2, out_cp = [], []
        for c in range(NCHUNK):
            rdma1[c].wait_recv()
            in_h[c].wait()
            off = h * half + c * rows
            red[pl.ds(c * rows, rows), :] = (
                xa_h[pl.ds(c * rows, rows), :]
                + s1_recv[pl.ds(c * rows, rows), :].astype(jnp.float32)
            ).astype(jnp.bfloat16)
            r = pltpu.make_async_remote_copy(
                src_ref=red.at[pl.ds(c * rows, rows), :],
                dst_ref=out_ref.at[pl.ds(off, rows), :],
                send_sem=sems2_s.at[c],
                recv_sem=sems2_r.at[c],
                device_id=(my_x, 1 - my_y),
                device_id_type=pl.DeviceIdType.MESH,
            )
            r.start()
            rdma2.append(r)
            cp = pltpu.make_async_copy(
                red.at[pl.ds(c * rows, rows), :],
                out_ref.at[pl.ds(off, rows), :],
                sems_out.at[c],
            )
            cp.start()
            out_cp.append(cp)

        for c in range(NCHUNK):
            rdma2[c].wait_recv()
        for c in range(NCHUNK):
            out_cp[c].wait()
            rdma1[c].wait_send()
            rdma2[c].wait_send()

    return pl.pallas_call(
        body,
        out_shape=jax.ShapeDtypeStruct((m, n), jnp.bfloat16),
        in_specs=[pl.BlockSpec(memory_space=pl.ANY)],
        out_specs=pl.BlockSpec(memory_space=pl.ANY),
        scratch_shapes=[
            pltpu.VMEM((half, n), jnp.float32),
            pltpu.VMEM((half, n), jnp.float32),
            pltpu.VMEM((half, n), jnp.bfloat16),
            pltpu.VMEM((half, n), jnp.bfloat16),
            pltpu.VMEM((half, n), jnp.bfloat16),
            pltpu.SemaphoreType.DMA((NCHUNK,)),
            pltpu.SemaphoreType.DMA((NCHUNK,)),
            pltpu.SemaphoreType.DMA((NCHUNK,)),
            pltpu.SemaphoreType.DMA((NCHUNK,)),
            pltpu.SemaphoreType.DMA((NCHUNK,)),
            pltpu.SemaphoreType.DMA((NCHUNK,)),
            pltpu.SemaphoreType.DMA((NCHUNK,)),
        ],
        compiler_params=pltpu.CompilerParams(collective_id=0),
    )(x)


# device time: 58997 ns/iter; 1.8587x vs baseline; 1.0432x over previous
import jax
import jax.numpy as jnp
from jax import lax
from jax.experimental import pallas as pl
from jax.experimental.pallas import tpu as pltpu

NCHUNK = 16


def kernel(x):
    m, n = x.shape
    half = m // 2
    rows = half // NCHUNK

    def body(x_ref, out_ref, xa_oh, xa_h, s1_send, s1_recv, red,
             sems_in_oh, sems_in_h, sems_out,
             sems1_s, sems1_r, sems2_s, sems2_r):
        my_x = lax.axis_index("x")
        my_y = lax.axis_index("y")
        h = my_x ^ my_y
        oh = 1 - h

        in_oh, in_h = [], []
        for c in range(NCHUNK):
            cp = pltpu.make_async_copy(
                x_ref.at[pl.ds(oh * half + c * rows, rows), :],
                xa_oh.at[pl.ds(c * rows, rows), :],
                sems_in_oh.at[c],
            )
            cp.start()
            in_oh.append(cp)
            cp = pltpu.make_async_copy(
                x_ref.at[pl.ds(h * half + c * rows, rows), :],
                xa_h.at[pl.ds(c * rows, rows), :],
                sems_in_h.at[c],
            )
            cp.start()
            in_h.append(cp)

        barrier_sem = pltpu.get_barrier_semaphore()
        for nbr in ((1 - my_x, my_y), (my_x, 1 - my_y)):
            pl.semaphore_signal(
                barrier_sem, inc=1,
                device_id=nbr, device_id_type=pl.DeviceIdType.MESH,
            )
        pl.semaphore_wait(barrier_sem, 2)

        rdma1 = []
        for c in range(NCHUNK):
            in_oh[c].wait()
            s1_send[pl.ds(c * rows, rows), :] = (
                xa_oh[pl.ds(c * rows, rows), :].astype(jnp.bfloat16)
            )
            r = pltpu.make_async_remote_copy(
                src_ref=s1_send.at[pl.ds(c * rows, rows), :],
                dst_ref=s1_recv.at[pl.ds(c * rows, rows), :],
                send_sem=sems1_s.at[c],
                recv_sem=sems1_r.at[c],
                device_id=(1 - my_x, my_y),
                device_id_type=pl.DeviceIdType.MESH,
            )
            r.start()
            rdma1.append(r)

        rdma2, out_cp = [], []
        for c in range(NCHUNK):
            rdma1[c].wait_recv()
            in_h[c].wait()
            off = h * half + c * rows
            red[pl.ds(c * rows, rows), :] = (
                xa_h[pl.ds(c * rows, rows), :]
                + s1_recv[pl.ds(c * rows, rows), :].astype(jnp.float32)
            ).astype(jnp.bfloat16)
            r = pltpu.make_async_remote_copy(
                src_ref=red.at[pl.ds(c * rows, rows), :],
                dst_ref=out_ref.at[pl.ds(off, rows), :],
                send_sem=sems2_s.at[c],
                recv_sem=sems2_r.at[c],
                device_id=(my_x, 1 - my_y),
                device_id_type=pl.DeviceIdType.MESH,
            )
            r.start()
            rdma2.append(r)
            cp = pltpu.make_async_copy(
                red.at[pl.ds(c * rows, rows), :],
                out_ref.at[pl.ds(off, rows), :],
                sems_out.at[c],
            )
            cp.start()
            out_cp.append(cp)

        for c in range(NCHUNK):
            rdma2[c].wait_recv()
        for c in range(NCHUNK):
            out_cp[c].wait()
            rdma1[c].wait_send()
            rdma2[c].wait_send()

    return pl.pallas_call(
        body,
        out_shape=jax.ShapeDtypeStruct((m, n), jnp.bfloat16),
        in_specs=[pl.BlockSpec(memory_space=pl.ANY)],
        out_specs=pl.BlockSpec(memory_space=pl.ANY),
        scratch_shapes=[
            pltpu.VMEM((half, n), jnp.float32),
            pltpu.VMEM((half, n), jnp.float32),
            pltpu.VMEM((half, n), jnp.bfloat16),
            pltpu.VMEM((half, n), jnp.bfloat16),
            pltpu.VMEM((half, n), jnp.bfloat16),
            pltpu.SemaphoreType.DMA((NCHUNK,)),
            pltpu.SemaphoreType.DMA((NCHUNK,)),
            pltpu.SemaphoreType.DMA((NCHUNK,)),
            pltpu.SemaphoreType.DMA((NCHUNK,)),
            pltpu.SemaphoreType.DMA((NCHUNK,)),
            pltpu.SemaphoreType.DMA((NCHUNK,)),
            pltpu.SemaphoreType.DMA((NCHUNK,)),
        ],
        compiler_params=pltpu.CompilerParams(collective_id=0),
    )(x)


# device time: 54328 ns/iter; 2.0184x vs baseline; 1.0859x over previous
import jax
import jax.numpy as jnp
from jax import lax
from jax.experimental import pallas as pl
from jax.experimental.pallas import tpu as pltpu

import os

NCHUNK = 16
_PROBE = int(os.environ.get("PROBE", "0"))


def kernel(x):
    m, n = x.shape
    half = m // 2
    rows = half // NCHUNK

    def body(x_ref, out_ref, xa_oh, xa_h, s1_send, s1_recv, red,
             sems_in_oh, sems_in_h, sems_out,
             sems1_s, sems1_r, sems2_s, sems2_r):
        my_x = lax.axis_index("x")
        my_y = lax.axis_index("y")
        h = my_x ^ my_y
        oh = 1 - h

        in_oh, in_h = [], []
        for c in range(NCHUNK):
            cp = pltpu.make_async_copy(
                x_ref.at[pl.ds(oh * half + c * rows, rows), :],
                xa_oh.at[pl.ds(c * rows, rows), :],
                sems_in_oh.at[c],
            )
            cp.start()
            in_oh.append(cp)
            cp = pltpu.make_async_copy(
                x_ref.at[pl.ds(h * half + c * rows, rows), :],
                xa_h.at[pl.ds(c * rows, rows), :],
                sems_in_h.at[c],
            )
            cp.start()
            in_h.append(cp)

        barrier_sem = pltpu.get_barrier_semaphore()
        for nbr in ((1 - my_x, my_y), (my_x, 1 - my_y)):
            pl.semaphore_signal(
                barrier_sem, inc=1,
                device_id=nbr, device_id_type=pl.DeviceIdType.MESH,
            )
        pl.semaphore_wait(barrier_sem, 2)

        rdma1 = []
        for c in range(NCHUNK):
            in_oh[c].wait()
            s1_send[pl.ds(c * rows, rows), :] = (
                xa_oh[pl.ds(c * rows, rows), :].astype(jnp.bfloat16)
            )
            if _PROBE != 2:
                r = pltpu.make_async_remote_copy(
                    src_ref=s1_send.at[pl.ds(c * rows, rows), :],
                    dst_ref=s1_recv.at[pl.ds(c * rows, rows), :],
                    send_sem=sems1_s.at[c],
                    recv_sem=sems1_r.at[c],
                    device_id=(1 - my_x, my_y),
                    device_id_type=pl.DeviceIdType.MESH,
                )
                r.start()
                rdma1.append(r)

        rdma2, out_cp = [], []
        for c in range(NCHUNK):
            if _PROBE != 2:
                rdma1[c].wait_recv()
            in_h[c].wait()
            off = h * half + c * rows
            red[pl.ds(c * rows, rows), :] = (
                xa_h[pl.ds(c * rows, rows), :]
                + s1_recv[pl.ds(c * rows, rows), :].astype(jnp.float32)
            ).astype(jnp.bfloat16)
            if _PROBE != 1:
                r = pltpu.make_async_remote_copy(
                    src_ref=red.at[pl.ds(c * rows, rows), :],
                    dst_ref=out_ref.at[pl.ds(off, rows), :],
                    send_sem=sems2_s.at[c],
                    recv_sem=sems2_r.at[c],
                    device_id=(my_x, 1 - my_y),
                    device_id_type=pl.DeviceIdType.MESH,
                )
                r.start()
                rdma2.append(r)
            cp = pltpu.make_async_copy(
                red.at[pl.ds(c * rows, rows), :],
                out_ref.at[pl.ds(off, rows), :],
                sems_out.at[c],
            )
            cp.start()
            out_cp.append(cp)

        for r in rdma2:
            r.wait_recv()
        for c in range(NCHUNK):
            out_cp[c].wait()
        for r in rdma1:
            r.wait_send()
        for r in rdma2:
            r.wait_send()

    return pl.pallas_call(
        body,
        out_shape=jax.ShapeDtypeStruct((m, n), jnp.bfloat16),
        in_specs=[pl.BlockSpec(memory_space=pl.ANY)],
        out_specs=pl.BlockSpec(memory_space=pl.ANY),
        scratch_shapes=[
            pltpu.VMEM((half, n), jnp.float32),
            pltpu.VMEM((half, n), jnp.float32),
            pltpu.VMEM((half, n), jnp.bfloat16),
            pltpu.VMEM((half, n), jnp.bfloat16),
            pltpu.VMEM((half, n), jnp.bfloat16),
            pltpu.SemaphoreType.DMA((NCHUNK,)),
            pltpu.SemaphoreType.DMA((NCHUNK,)),
            pltpu.SemaphoreType.DMA((NCHUNK,)),
            pltpu.SemaphoreType.DMA((NCHUNK,)),
            pltpu.SemaphoreType.DMA((NCHUNK,)),
            pltpu.SemaphoreType.DMA((NCHUNK,)),
            pltpu.SemaphoreType.DMA((NCHUNK,)),
        ],
        compiler_params=pltpu.CompilerParams(collective_id=0),
    )(x)
